# baseline (speedup 1.0000x reference)
import functools

import jax
import jax.numpy as jnp
from jax import lax
from jax.experimental import pallas as pl
from jax.experimental.pallas import tpu as pltpu
from jax.experimental.pallas import tpu_sc as plsc

_B, _S, _D, _V = 4, 2048, 1024, 100000
_NC, _NS = 2, 16
_NW = _NC * _NS
_WS = _S // _NW
_C = 16
_CPB = _WS // _C
_NCH = _B * _CPB


def _emb_body(idx_hbm, tok_hbm, wpe_hbm, out_hbm,
              idx_v, wpe_v, tok0, tok1,
              gsem0, gsem1, osem0, osem1, isem, wsem):
    toks = [tok0, tok1]
    gsems = [gsem0, gsem1]
    osems = [osem0, osem1]

    wid = lax.axis_index("s") * _NC + lax.axis_index("c")
    s_base = wid * _WS
    icps = [pltpu.async_copy(idx_hbm.at[pl.ds(b * _S + s_base, _WS)],
                             idx_v.at[pl.ds(b * _WS, _WS)], isem)
            for b in range(_B)]
    wcp = pltpu.async_copy(wpe_hbm.at[pl.ds(s_base, _WS)], wpe_v, wsem)
    for cp in icps:
        cp.wait()
    wcp.wait()

    gcp = [None, None]
    ocp = [None, None]

    def issue(t):
        p = t % 2
        gcp[p] = pltpu.async_copy(
            tok_hbm.at[idx_v.at[pl.ds(t * _C, _C)]], toks[p], gsems[p])

    def finish(t):
        p = t % 2
        b, cc = t // _CPB, t % _CPB
        gcp[p].wait()

        def _add_row(r, carry):
            for j in range(_D // 16):
                sl = pl.ds(j * 16, 16)
                toks[p][r, sl] = toks[p][r, sl] + wpe_v[cc * _C + r, sl]
            return carry

        lax.fori_loop(0, _C, _add_row, 0)
        ocp[p] = pltpu.async_copy(
            toks[p], out_hbm.at[pl.ds(b * _S + s_base + cc * _C, _C)],
            osems[p])

    issue(0)
    for t in range(1, _NCH):
        p = t % 2
        if ocp[p] is not None:
            ocp[p].wait()
        issue(t)
        finish(t - 1)
    finish(_NCH - 1)
    ocp[0].wait()
    ocp[1].wait()


_sc_embed = functools.partial(
    pl.kernel,
    out_type=jax.ShapeDtypeStruct((_B * _S, _D), jnp.float32),
    mesh=plsc.VectorSubcoreMesh(core_axis_name="c", subcore_axis_name="s"),
    scratch_types=[
        pltpu.VMEM((_B * _WS,), jnp.int32),
        pltpu.VMEM((_WS, _D), jnp.float32),
        pltpu.VMEM((_C, _D), jnp.float32),
        pltpu.VMEM((_C, _D), jnp.float32),
        pltpu.SemaphoreType.DMA,
        pltpu.SemaphoreType.DMA,
        pltpu.SemaphoreType.DMA,
        pltpu.SemaphoreType.DMA,
        pltpu.SemaphoreType.DMA,
        pltpu.SemaphoreType.DMA,
    ],
)(_emb_body)


def kernel(idx, tok_emb, wpe):
    flat = _sc_embed(idx.reshape(_B * _S), tok_emb, wpe)
    return flat.reshape(_B, _S, _D)


# speedup vs baseline: 1.2273x; 1.2273x over previous
import functools

import jax
import jax.numpy as jnp
from jax import lax
from jax.experimental import pallas as pl
from jax.experimental.pallas import tpu as pltpu
from jax.experimental.pallas import tpu_sc as plsc

_B, _S, _D, _V = 4, 2048, 1024, 100000
_NC, _NS = 2, 16
_NW = _NC * _NS
_RPW = (_B * _S) // _NW
_C = 16
_NCH = _RPW // _C
_NB = 3


def _emb_body(idx_hbm, tok_hbm, wpe_hbm, out_hbm, idx_v,
              tok0, tok1, tok2, wpe0, wpe1, wpe2,
              g0, g1, g2, w0, w1, w2, o0, o1, o2):
    toks = [tok0, tok1, tok2]
    wpes = [wpe0, wpe1, wpe2]
    gsems = [g0, g1, g2]
    wsems = [w0, w1, w2]
    osems = [o0, o1, o2]

    wid = lax.axis_index("s") * _NC + lax.axis_index("c")
    base = wid * _RPW
    s0 = lax.rem(base, _S)
    pltpu.sync_copy(idx_hbm.at[pl.ds(base, _RPW)], idx_v)

    gcp = [None] * _NB
    wcp = [None] * _NB
    ocp = [None] * _NB

    def issue(t):
        p = t % _NB
        gcp[p] = pltpu.async_copy(
            tok_hbm.at[idx_v.at[pl.ds(t * _C, _C)]], toks[p], gsems[p])
        wcp[p] = pltpu.async_copy(
            wpe_hbm.at[pl.ds(s0 + t * _C, _C)], wpes[p], wsems[p])

    def finish(t):
        p = t % _NB
        gcp[p].wait()
        wcp[p].wait()

        def _add_row(r, carry):
            for j in range(_D // 16):
                sl = pl.ds(j * 16, 16)
                toks[p][r, sl] = toks[p][r, sl] + wpes[p][r, sl]
            return carry

        lax.fori_loop(0, _C, _add_row, 0)
        ocp[p] = pltpu.async_copy(
            toks[p], out_hbm.at[pl.ds(base + t * _C, _C)], osems[p])

    issue(0)
    issue(1)
    for t in range(2, _NCH):
        p = t % _NB
        if ocp[p] is not None:
            ocp[p].wait()
        issue(t)
        finish(t - 2)
    finish(_NCH - 2)
    finish(_NCH - 1)
    for p in range(_NB):
        ocp[p].wait()


_sc_embed = functools.partial(
    pl.kernel,
    out_type=jax.ShapeDtypeStruct((_B * _S, _D), jnp.float32),
    mesh=plsc.VectorSubcoreMesh(core_axis_name="c", subcore_axis_name="s"),
    scratch_types=(
        [pltpu.VMEM((_RPW,), jnp.int32)]
        + [pltpu.VMEM((_C, _D), jnp.float32)] * (2 * _NB)
        + [pltpu.SemaphoreType.DMA] * (3 * _NB)
    ),
)(_emb_body)


def kernel(idx, tok_emb, wpe):
    flat = _sc_embed(idx.reshape(_B * _S), tok_emb, wpe)
    return flat.reshape(_B, _S, _D)
